# baseline (speedup 1.0000x reference)
import functools

import jax
import jax.numpy as jnp
from jax import lax
from jax.experimental import pallas as pl
from jax.experimental.pallas import tpu as pltpu
from jax.experimental.pallas import tpu_sc as plsc

C = 80
Z_OUT, H_OUT, W_OUT = 1, 256, 256
N_OUT = Z_OUT * H_OUT * W_OUT
R = 1024
NRANGES = N_OUT // R
K = 256
KI = 128
NW = 32
RANGES_PER_W = NRANGES // NW
BOUNDS_PAD = 80


def _sc_body(depth_hbm, feat_hbm, rd_hbm, rf_hbm, rb_hbm, bounds_hbm, out_hbm,
             bounds_v,
             rd0, rd1, rd2, rf0, rf1, rf2, rb0, rb1, rb2,
             dg0, dg1, fg0, fg1, acc,
             si0, si1, si2, sd0, sd1, sf0, sf1):
    n_pts = rd_hbm.shape[0]
    off_max = n_pts - K
    rd_v = (rd0, rd1, rd2)
    rf_v = (rf0, rf1, rf2)
    rb_v = (rb0, rb1, rb2)
    dg_v = (dg0, dg1)
    fg_v = (fg0, fg1)
    si = (si0, si1, si2)
    sd = (sd0, sd1)
    sf = (sf0, sf1)

    def issue_idx(off, i):
        pltpu.async_copy(rd_hbm.at[pl.ds(off, K)], rd_v[i], si[i])
        pltpu.async_copy(rf_hbm.at[pl.ds(off, K)], rf_v[i], si[i])
        pltpu.async_copy(rb_hbm.at[pl.ds(off, K)], rb_v[i], si[i])

    def wait_idx(i):
        pltpu.make_async_copy(rd_hbm.at[pl.ds(0, K)], rd_v[i], si[i]).wait()
        pltpu.make_async_copy(rf_hbm.at[pl.ds(0, K)], rf_v[i], si[i]).wait()
        pltpu.make_async_copy(rb_hbm.at[pl.ds(0, K)], rb_v[i], si[i]).wait()

    def issue_gather(i3, i2):
        for h in range(K // KI):
            hs = pl.ds(h * KI, KI)
            pltpu.async_copy(depth_hbm.at[rd_v[i3].at[hs]],
                             dg_v[i2].at[hs], sd[i2])
            pltpu.async_copy(feat_hbm.at[rf_v[i3].at[hs]],
                             fg_v[i2].at[hs], sf[i2])

    def wait_gather(i3, i2):
        for h in range(K // KI):
            hs = pl.ds(h * KI, KI)
            pltpu.make_async_copy(depth_hbm.at[rd_v[i3].at[hs]],
                                  dg_v[i2].at[hs], sd[i2]).wait()
            pltpu.make_async_copy(feat_hbm.at[rf_v[i3].at[hs]],
                                  fg_v[i2].at[hs], sf[i2]).wait()

    wid = lax.axis_index("s") * 2 + lax.axis_index("c")
    pltpu.sync_copy(bounds_hbm, bounds_v)
    zero16 = jnp.zeros((16,), jnp.float32)

    def range_body(rg, _):
        j = wid * RANGES_PER_W + rg
        base = j * R
        sv = bounds_v[pl.ds(j, 16)]
        s = sv[0]
        e = sv[1]
        a = (s // 8) * 8
        nblk = (e - a + K - 1) // K

        @plsc.parallel_loop(0, R * C // 16, unroll=8)
        def zacc(i):
            acc[pl.ds(i * 16, 16)] = zero16

        def compute(off, u, i3, i2):
            fg = fg_v[i2]

            def grp(g, _):
                p16 = g * 16
                iota16 = lax.iota(jnp.int32, 16)
                bv16 = rb_v[i3][pl.ds(p16, 16)]
                wd16 = dg_v[i2][pl.ds(p16, 16)]
                pg16 = off + p16 + iota16
                ok = jnp.logical_and(
                    jnp.logical_and(pg16 < e, pg16 >= u), bv16 >= base)
                wv16 = jnp.where(ok, wd16, zero16)
                loc16 = jnp.clip(bv16 - base, 0, R - 1)
                prow = p16 + iota16
                bl = loc16 * C

                @plsc.parallel_loop(0, C, unroll=8)
                def step(m):
                    c_hi = m - jnp.bitwise_and(m, 15)
                    col = jnp.bitwise_and(iota16 + m, 15) + c_hi
                    v = wv16 * plsc.load_gather(fg, [prow, col])
                    plsc.addupdate_scatter(acc, [bl + col], v)
                return 0
            lax.fori_loop(0, K // 16, grp, 0)

        issue_idx(jnp.minimum(a, off_max), 0)
        issue_idx(jnp.minimum(a + K, off_max), 1)
        wait_idx(0)
        issue_gather(0, 0)

        def round_body(tr, _):
            t0 = tr * 6
            for k in range(6):
                t = t0 + k
                issue_idx(jnp.minimum(a + (t + 2) * K, off_max), (k + 2) % 3)
                wait_idx((k + 1) % 3)
                issue_gather((k + 1) % 3, (k + 1) % 2)
                wait_gather(k % 3, k % 2)
                u = a + t * K
                compute(jnp.minimum(u, off_max), u, k % 3, k % 2)
            return 0
        nround = (nblk + 5) // 6
        lax.fori_loop(0, nround, round_body, 0)
        wait_idx(1)
        wait_gather(0, 0)

        pltpu.sync_copy(acc, out_hbm.at[pl.ds(base * C, R * C)])
        return 0
    lax.fori_loop(0, RANGES_PER_W, range_body, 0)


@jax.jit
def _bev_pool(depth_flat, feat_flat, rd_p, rf_p, rb_p, bounds):
    call = functools.partial(
        pl.kernel,
        out_type=jax.ShapeDtypeStruct((N_OUT * C,), jnp.float32),
        mesh=plsc.VectorSubcoreMesh(core_axis_name="c", subcore_axis_name="s"),
        compiler_params=pltpu.CompilerParams(
            use_tc_tiling_on_sc=False, needs_layout_passes=False),
        scratch_types=[
            pltpu.VMEM((BOUNDS_PAD,), jnp.int32),
            pltpu.VMEM((K,), jnp.int32),
            pltpu.VMEM((K,), jnp.int32),
            pltpu.VMEM((K,), jnp.int32),
            pltpu.VMEM((K,), jnp.int32),
            pltpu.VMEM((K,), jnp.int32),
            pltpu.VMEM((K,), jnp.int32),
            pltpu.VMEM((K,), jnp.int32),
            pltpu.VMEM((K,), jnp.int32),
            pltpu.VMEM((K,), jnp.int32),
            pltpu.VMEM((K,), jnp.float32),
            pltpu.VMEM((K,), jnp.float32),
            pltpu.VMEM((K, C), jnp.float32),
            pltpu.VMEM((K, C), jnp.float32),
            pltpu.VMEM((R * C,), jnp.float32),
            pltpu.SemaphoreType.DMA,
            pltpu.SemaphoreType.DMA,
            pltpu.SemaphoreType.DMA,
            pltpu.SemaphoreType.DMA,
            pltpu.SemaphoreType.DMA,
            pltpu.SemaphoreType.DMA,
            pltpu.SemaphoreType.DMA,
        ],
    )(_sc_body)
    return call(depth_flat, feat_flat, rd_p, rf_p, rb_p, bounds)


def kernel(depth, feat, ranks_depth, ranks_feat, ranks_bev,
           interval_starts, interval_lengths):
    b = feat.shape[0]
    c = feat.shape[2]
    depth_flat = depth.reshape(-1)
    feat_flat = jnp.transpose(feat, (0, 1, 3, 4, 2)).reshape(-1, c)
    qs = jnp.arange(NRANGES + 1, dtype=jnp.int32) * R
    n_pts = ranks_bev.shape[0]
    lo = jnp.zeros((NRANGES + 1,), jnp.int32)
    hi = jnp.full((NRANGES + 1,), n_pts, jnp.int32)
    nsteps = max(1, (n_pts - 1).bit_length())
    for _ in range(nsteps):
        mid = (lo + hi) // 2
        pred = jnp.logical_and(mid < n_pts,
                               jnp.take(ranks_bev, mid, mode='clip') < qs)
        lo = jnp.where(pred, mid + 1, lo)
        hi = jnp.where(pred, hi, mid)
    bounds = lo
    bounds = jnp.concatenate(
        [bounds, jnp.zeros((BOUNDS_PAD - (NRANGES + 1),), jnp.int32)])
    out = _bev_pool(depth_flat, feat_flat, ranks_depth, ranks_feat,
                    ranks_bev, bounds)
    out = out.reshape(b, Z_OUT, H_OUT, W_OUT, c)
    return jnp.transpose(out, (0, 4, 1, 2, 3))


# speedup vs baseline: 1.0347x; 1.0347x over previous
import functools

import jax
import jax.numpy as jnp
from jax import lax
from jax.experimental import pallas as pl
from jax.experimental.pallas import tpu as pltpu
from jax.experimental.pallas import tpu_sc as plsc

C = 80
Z_OUT, H_OUT, W_OUT = 1, 256, 256
N_OUT = Z_OUT * H_OUT * W_OUT
R = 1024
NRANGES = N_OUT // R
K = 128
NW = 32
RANGES_PER_W = NRANGES // NW
BOUNDS_PAD = 80


def _sc_body(depth_hbm, feat_hbm, rd_hbm, rf_hbm, rb_hbm, bounds_hbm, out_hbm,
             bounds_v,
             rd0, rd1, rd2, rf0, rf1, rf2, rb0, rb1, rb2,
             dg0, dg1, fg0, fg1, acc,
             si0, si1, si2, sd0, sd1, sf0, sf1):
    n_pts = rd_hbm.shape[0]
    off_max = n_pts - K
    rd_v = (rd0, rd1, rd2)
    rf_v = (rf0, rf1, rf2)
    rb_v = (rb0, rb1, rb2)
    dg_v = (dg0, dg1)
    fg_v = (fg0, fg1)
    si = (si0, si1, si2)
    sd = (sd0, sd1)
    sf = (sf0, sf1)

    def issue_idx(off, i):
        pltpu.async_copy(rd_hbm.at[pl.ds(off, K)], rd_v[i], si[i])
        pltpu.async_copy(rf_hbm.at[pl.ds(off, K)], rf_v[i], si[i])
        pltpu.async_copy(rb_hbm.at[pl.ds(off, K)], rb_v[i], si[i])

    def wait_idx(i):
        pltpu.make_async_copy(rd_hbm.at[pl.ds(0, K)], rd_v[i], si[i]).wait()
        pltpu.make_async_copy(rf_hbm.at[pl.ds(0, K)], rf_v[i], si[i]).wait()
        pltpu.make_async_copy(rb_hbm.at[pl.ds(0, K)], rb_v[i], si[i]).wait()

    def issue_gather(i3, i2):
        pltpu.async_copy(depth_hbm.at[rd_v[i3]], dg_v[i2], sd[i2])
        pltpu.async_copy(feat_hbm.at[rf_v[i3]], fg_v[i2], sf[i2])

    def wait_gather(i3, i2):
        pltpu.make_async_copy(depth_hbm.at[rd_v[i3]], dg_v[i2], sd[i2]).wait()
        pltpu.make_async_copy(feat_hbm.at[rf_v[i3]], fg_v[i2], sf[i2]).wait()

    wid = lax.axis_index("s") * 2 + lax.axis_index("c")
    pltpu.sync_copy(bounds_hbm, bounds_v)
    zero16 = jnp.zeros((16,), jnp.float32)

    def range_body(rg, _):
        j = wid * RANGES_PER_W + rg
        base = j * R
        sv = bounds_v[pl.ds(j, 16)]
        s = sv[0]
        e = sv[1]
        a = (s // 8) * 8
        nblk = (e - a + K - 1) // K

        @plsc.parallel_loop(0, R * C // 16, unroll=8)
        def zacc(i):
            acc[pl.ds(i * 16, 16)] = zero16

        def compute(off, u, i3, i2):
            fg = fg_v[i2]

            def grp(g, _):
                p16 = g * 16
                iota16 = lax.iota(jnp.int32, 16)
                bv16 = rb_v[i3][pl.ds(p16, 16)]
                wd16 = dg_v[i2][pl.ds(p16, 16)]
                pg16 = off + p16 + iota16
                ok = jnp.logical_and(
                    jnp.logical_and(pg16 < e, pg16 >= u), bv16 >= base)
                wv16 = jnp.where(ok, wd16, zero16)
                loc16 = jnp.clip(bv16 - base, 0, R - 1)
                prow = p16 + iota16
                bl = loc16 * C

                @plsc.parallel_loop(0, C, unroll=8)
                def step(m):
                    c_hi = m - jnp.bitwise_and(m, 15)
                    col = jnp.bitwise_and(iota16 + m, 15) + c_hi
                    v = wv16 * plsc.load_gather(fg, [prow, col])
                    plsc.addupdate_scatter(acc, [bl + col], v)
                return 0
            lax.fori_loop(0, K // 16, grp, 0)

        issue_idx(jnp.minimum(a, off_max), 0)
        issue_idx(jnp.minimum(a + K, off_max), 1)
        wait_idx(0)
        issue_gather(0, 0)

        def round_body(tr, _):
            t0 = tr * 6
            for k in range(6):
                t = t0 + k
                issue_idx(jnp.minimum(a + (t + 2) * K, off_max), (k + 2) % 3)
                wait_idx((k + 1) % 3)
                issue_gather((k + 1) % 3, (k + 1) % 2)
                wait_gather(k % 3, k % 2)
                u = a + t * K
                compute(jnp.minimum(u, off_max), u, k % 3, k % 2)
            return 0
        nround = (nblk + 5) // 6
        lax.fori_loop(0, nround, round_body, 0)
        wait_idx(1)
        wait_gather(0, 0)

        pltpu.sync_copy(acc, out_hbm.at[pl.ds(base * C, R * C)])
        return 0
    lax.fori_loop(0, RANGES_PER_W, range_body, 0)


@jax.jit
def _bev_pool(depth_flat, feat_flat, rd_p, rf_p, rb_p, bounds):
    call = functools.partial(
        pl.kernel,
        out_type=jax.ShapeDtypeStruct((N_OUT * C,), jnp.float32),
        mesh=plsc.VectorSubcoreMesh(core_axis_name="c", subcore_axis_name="s"),
        compiler_params=pltpu.CompilerParams(
            use_tc_tiling_on_sc=False, needs_layout_passes=False),
        scratch_types=[
            pltpu.VMEM((BOUNDS_PAD,), jnp.int32),
            pltpu.VMEM((K,), jnp.int32),
            pltpu.VMEM((K,), jnp.int32),
            pltpu.VMEM((K,), jnp.int32),
            pltpu.VMEM((K,), jnp.int32),
            pltpu.VMEM((K,), jnp.int32),
            pltpu.VMEM((K,), jnp.int32),
            pltpu.VMEM((K,), jnp.int32),
            pltpu.VMEM((K,), jnp.int32),
            pltpu.VMEM((K,), jnp.int32),
            pltpu.VMEM((K,), jnp.float32),
            pltpu.VMEM((K,), jnp.float32),
            pltpu.VMEM((K, C), jnp.float32),
            pltpu.VMEM((K, C), jnp.float32),
            pltpu.VMEM((R * C,), jnp.float32),
            pltpu.SemaphoreType.DMA,
            pltpu.SemaphoreType.DMA,
            pltpu.SemaphoreType.DMA,
            pltpu.SemaphoreType.DMA,
            pltpu.SemaphoreType.DMA,
            pltpu.SemaphoreType.DMA,
            pltpu.SemaphoreType.DMA,
        ],
    )(_sc_body)
    return call(depth_flat, feat_flat, rd_p, rf_p, rb_p, bounds)


def kernel(depth, feat, ranks_depth, ranks_feat, ranks_bev,
           interval_starts, interval_lengths):
    b = feat.shape[0]
    c = feat.shape[2]
    depth_flat = depth.reshape(-1)
    feat_flat = jnp.transpose(feat, (0, 1, 3, 4, 2)).reshape(-1, c)
    qs = jnp.arange(NRANGES + 1, dtype=jnp.int32) * R
    n_pts = ranks_bev.shape[0]
    lo = jnp.zeros((NRANGES + 1,), jnp.int32)
    hi = jnp.full((NRANGES + 1,), n_pts, jnp.int32)
    nsteps = max(1, (n_pts - 1).bit_length())
    for _ in range(nsteps):
        mid = (lo + hi) // 2
        pred = jnp.logical_and(mid < n_pts,
                               jnp.take(ranks_bev, mid, mode='clip') < qs)
        lo = jnp.where(pred, mid + 1, lo)
        hi = jnp.where(pred, hi, mid)
    bounds = lo
    bounds = jnp.concatenate(
        [bounds, jnp.zeros((BOUNDS_PAD - (NRANGES + 1),), jnp.int32)])
    out = _bev_pool(depth_flat, feat_flat, ranks_depth, ranks_feat,
                    ranks_bev, bounds)
    out = out.reshape(b, Z_OUT, H_OUT, W_OUT, c)
    return jnp.transpose(out, (0, 4, 1, 2, 3))
